# Initial kernel scaffold; baseline (speedup 1.0000x reference)
import jax
import jax.numpy as jnp
from jax.experimental import pallas as pl


def kernel(ent_adj, rel_adj, node_size, rel_size, adj_list, r_index, r_val, triple_size, mask, ent_emb, rel_emb, e_kernels, r_kernels):
    raise NotImplementedError("write your pallas kernel here")


# speedup vs baseline: 14.9637x; 14.9637x over previous
import functools

import jax
import jax.numpy as jnp
from jax import lax
from jax.experimental import pallas as pl
from jax.experimental.pallas import tpu as pltpu
from jax.experimental.pallas import tpu_sc as plsc

NC = 2
NS = 16
CH = 128
D = 128

_MESH = plsc.VectorSubcoreMesh(core_axis_name="c", subcore_axis_name="s")


def _dual_segsum(tbl_a, tbl_b, col_a, row_a, col_b, row_b, n_rows):
    E = col_a.shape[0]
    nch = E // CH
    niter = (nch + NS - 1) // NS
    n_pad = -(-n_rows // (8 * NS)) * (8 * NS)
    rpw = n_pad // NS

    @functools.partial(
        pl.kernel,
        out_type=jax.ShapeDtypeStruct((2 * n_pad, D), jnp.float32),
        mesh=_MESH,
        scratch_types=dict(
            acc=pltpu.VMEM_SHARED((n_pad, D), jnp.float32),
            cidx=pltpu.VMEM((CH,), jnp.int32),
            ridx=pltpu.VMEM((CH,), jnp.int32),
            rows=pltpu.VMEM((CH, D), jnp.float32),
            sem=pltpu.SemaphoreType.DMA,
        ),
    )
    def k(ta, tb, ca, ra, cb, rb, zeros, out, acc, cidx, ridx, rows, sem):
        c = lax.axis_index("c")
        s = lax.axis_index("s")
        pltpu.sync_copy(zeros.at[pl.ds(s * rpw, rpw)], acc.at[pl.ds(s * rpw, rpw)])
        plsc.subcore_barrier()

        def run(col_h, row_h, tbl_h):
            def body(g, carry):
                k_id = g * NS + s

                @pl.when(k_id < nch)
                def _():
                    base = k_id * CH
                    pltpu.sync_copy(col_h.at[pl.ds(base, CH)], cidx)
                    pltpu.sync_copy(row_h.at[pl.ds(base, CH)], ridx)
                    pltpu.async_copy(tbl_h.at[cidx], rows, sem).wait()
                    pltpu.sync_copy(rows, acc.at[ridx], add=True)

                return carry

            lax.fori_loop(0, niter, body, 0)

        @pl.when(c == 0)
        def _():
            run(ca, ra, ta)

        @pl.when(c == 1)
        def _():
            run(cb, rb, tb)
        plsc.subcore_barrier()
        pltpu.sync_copy(acc.at[pl.ds(s * rpw, rpw)],
                        out.at[pl.ds(c * n_pad + s * rpw, rpw)])

    zeros = jnp.zeros((n_pad, D), jnp.float32)
    out = k(tbl_a, tbl_b, col_a, row_a, col_b, row_b, zeros)
    return out[:n_rows], out[n_pad : n_pad + n_rows]


def _rel_combine(rel_emb, col, row, val, n_rows):
    E = col.shape[0]
    NW = NC * NS
    nch = E // CH
    niter = (nch + NW - 1) // NW
    n_pad = -(-n_rows // (8 * NS)) * (8 * NS)
    rpw = n_pad // NS

    @functools.partial(
        pl.kernel,
        out_type=jax.ShapeDtypeStruct((2 * n_pad, D), jnp.float32),
        mesh=_MESH,
        scratch_types=dict(
            acc=pltpu.VMEM_SHARED((n_pad, D), jnp.float32),
            cidx=pltpu.VMEM((CH,), jnp.int32),
            ridx=pltpu.VMEM((CH,), jnp.int32),
            vals=pltpu.VMEM((CH,), jnp.float32),
            rows=pltpu.VMEM((CH, D), jnp.float32),
            sem=pltpu.SemaphoreType.DMA,
        ),
    )
    def k(tbl, ch, rh, vh, zeros, out, acc, cidx, ridx, vals, rows, sem):
        c = lax.axis_index("c")
        s = lax.axis_index("s")
        w = s * NC + c
        pltpu.sync_copy(zeros.at[pl.ds(s * rpw, rpw)], acc.at[pl.ds(s * rpw, rpw)])
        plsc.subcore_barrier()

        def body(g, carry):
            k_id = g * NW + w

            @pl.when(k_id < nch)
            def _():
                base = k_id * CH
                pltpu.sync_copy(ch.at[pl.ds(base, CH)], cidx)
                pltpu.sync_copy(rh.at[pl.ds(base, CH)], ridx)
                pltpu.sync_copy(vh.at[pl.ds(base, CH)], vals)
                pltpu.async_copy(tbl.at[cidx], rows, sem).wait()

                def mul(q, cc):
                    v16 = vals[pl.ds(q * 16, 16)]
                    for e16 in range(16):
                        v = v16[e16]
                        e = q * 16 + e16
                        for j in range(D // 16):
                            sl = pl.ds(j * 16, 16)
                            rows[e, sl] = rows[e, sl] * v
                    return cc

                lax.fori_loop(0, CH // 16, mul, 0)
                pltpu.sync_copy(rows, acc.at[ridx], add=True)

            return carry

        lax.fori_loop(0, niter, body, 0)
        plsc.subcore_barrier()
        pltpu.sync_copy(acc.at[pl.ds(s * rpw, rpw)],
                        out.at[pl.ds(c * n_pad + s * rpw, rpw)])

    zeros = jnp.zeros((n_pad, D), jnp.float32)
    out = k(rel_emb, col, row, val, zeros)
    return out[:n_rows] + out[n_pad : n_pad + n_rows]


def _degrees(ent_row, rel_row, adj_row, n_rows):
    E = ent_row.shape[0]
    NW = NC * NS
    nch = E // CH
    niter_c = (nch + NS - 1) // NS
    niter_w = (nch + NW - 1) // NW
    n_pad = -(-n_rows // (8 * NS)) * (8 * NS)
    rpw = n_pad // NS

    @functools.partial(
        pl.kernel,
        out_type=jax.ShapeDtypeStruct((4 * n_pad, D), jnp.float32),
        mesh=_MESH,
        scratch_types=dict(
            acc=pltpu.VMEM_SHARED((n_pad, D), jnp.float32),
            ridx=pltpu.VMEM((CH,), jnp.int32),
            ones_v=pltpu.VMEM((CH, D), jnp.float32),
        ),
    )
    def k(eh, rh, ah, zeros, out, acc, ridx, ones_v):
        c = lax.axis_index("c")
        s = lax.axis_index("s")
        w = s * NC + c

        def fill(r, carry):
            for j in range(D // 16):
                ones_v[r, pl.ds(j * 16, 16)] = jnp.ones((16,), jnp.float32)
            return carry

        lax.fori_loop(0, CH, fill, 0)
        pltpu.sync_copy(zeros.at[pl.ds(s * rpw, rpw)], acc.at[pl.ds(s * rpw, rpw)])
        plsc.subcore_barrier()

        def count(rows_h, stride, first):
            def body(g, carry):
                k_id = g * stride + first

                @pl.when(k_id < nch)
                def _():
                    pltpu.sync_copy(rows_h.at[pl.ds(k_id * CH, CH)], ridx)
                    pltpu.sync_copy(ones_v, acc.at[ridx], add=True)

                return carry

            lax.fori_loop(0, (nch + stride - 1) // stride, body, 0)

        @pl.when(c == 0)
        def _():
            count(eh, NS, s)

        @pl.when(c == 1)
        def _():
            count(rh, NS, s)

        plsc.subcore_barrier()
        pltpu.sync_copy(acc.at[pl.ds(s * rpw, rpw)],
                        out.at[pl.ds(c * n_pad + s * rpw, rpw)])
        pltpu.sync_copy(zeros.at[pl.ds(s * rpw, rpw)], acc.at[pl.ds(s * rpw, rpw)])
        plsc.subcore_barrier()
        count(ah, NW, w)
        plsc.subcore_barrier()
        pltpu.sync_copy(acc.at[pl.ds(s * rpw, rpw)],
                        out.at[pl.ds((2 + c) * n_pad + s * rpw, rpw)])

    zeros = jnp.zeros((n_pad, D), jnp.float32)
    out = k(ent_row, rel_row, adj_row, zeros)
    deg_ent = out[:n_rows, 0]
    deg_rel = out[n_pad : n_pad + n_rows, 0]
    deg_adj = (out[2 * n_pad : 2 * n_pad + n_rows, 0]
               + out[3 * n_pad : 3 * n_pad + n_rows, 0])
    return deg_ent, deg_rel, deg_adj


def kernel(ent_adj, rel_adj, node_size, rel_size, adj_list, r_index, r_val,
           triple_size, mask, ent_emb, rel_emb, e_kernels, r_kernels):
    N = mask.shape[0]
    NR = rel_emb.shape[0]
    E = adj_list.shape[1]

    i32 = jnp.int32
    ent_row = ent_adj[0].astype(i32)
    ent_col = ent_adj[1].astype(i32)
    rel_row = rel_adj[0].astype(i32)
    rel_col = rel_adj[1].astype(i32)
    adj_row = adj_list[0].astype(i32)
    adj_col = adj_list[1].astype(i32)
    r_seg = r_index[0].astype(i32)
    r_rel = r_index[1].astype(i32)

    deg_ent, deg_rel, deg_adj = _degrees(ent_row, rel_row, adj_row, N)

    s_ent, s_rel = _dual_segsum(ent_emb, rel_emb, ent_col, ent_row, rel_col, rel_row, N)
    ent_feature = s_ent / (deg_ent + 1e-12)[:, None]
    rel_feature = s_rel / (deg_rel + 1e-12)[:, None]

    Rm = _rel_combine(rel_emb, r_rel, r_seg, r_val, NR)
    Rn = Rm / (jnp.linalg.norm(Rm, axis=1, keepdims=True) + 1e-12)

    row_s = adj_row[:NR]
    col_s = adj_col[:NR]

    def fixup(f, S, kvec):
        a = (Rn @ kvec)[:, 0]
        Ev = jnp.exp(a)
        G = f[col_s]
        dd = jnp.sum(G * Rn, axis=1)
        reflterm = Ev[:, None] * G - (2.0 * Ev * dd)[:, None] * Rn
        P = jax.ops.segment_sum(G, row_s, num_segments=N)
        Qn = jax.ops.segment_sum(reflterm, row_s, num_segments=N)
        cnt = jax.ops.segment_sum(jnp.ones((NR,), jnp.float32), row_s, num_segments=N)
        sE = jax.ops.segment_sum(Ev, row_s, num_segments=N)
        denom = (deg_adj - cnt) + sE
        return jnp.tanh((S - P + Qn) / (denom + 1e-12)[:, None])

    f_e = jnp.tanh(ent_feature)
    f_r = jnp.tanh(rel_feature)
    outs = [f_e, None, None, f_r, None, None]
    for l in range(2):
        s_e, s_r = _dual_segsum(f_e, f_r, adj_col, adj_row, adj_col, adj_row, N)
        f_e = fixup(f_e, s_e, e_kernels[l])
        f_r = fixup(f_r, s_r, r_kernels[l])
        outs[1 + l] = f_e
        outs[4 + l] = f_r

    return jnp.concatenate(outs, axis=-1)


# speedup vs baseline: 19.4374x; 1.2990x over previous
import functools

import jax
import jax.numpy as jnp
from jax import lax
from jax.experimental import pallas as pl
from jax.experimental.pallas import tpu as pltpu
from jax.experimental.pallas import tpu_sc as plsc

NC = 2
NS = 16
CH = 128
D = 128

_MESH = plsc.VectorSubcoreMesh(core_axis_name="c", subcore_axis_name="s")


def _dual_segsum(tbl_a, tbl_b, col_a, row_a, col_b, row_b, n_rows):
    E = col_a.shape[0]
    nch = E // CH
    niter = (nch + NS - 1) // NS
    n_pad = -(-n_rows // (8 * NS)) * (8 * NS)
    rpw = n_pad // NS

    @functools.partial(
        pl.kernel,
        out_type=jax.ShapeDtypeStruct((2 * n_pad, D), jnp.float32),
        mesh=_MESH,
        scratch_types=dict(
            acc=pltpu.VMEM_SHARED((n_pad, D), jnp.float32),
            cidx=pltpu.VMEM((2, CH), jnp.int32),
            ridx=pltpu.VMEM((2, CH), jnp.int32),
            rows=pltpu.VMEM((2, CH, D), jnp.float32),
            gsem=pltpu.SemaphoreType.DMA,
            ssem=pltpu.SemaphoreType.DMA,
        ),
    )
    def k(ta, tb, ca, ra, cb, rb, zeros, out, acc, cidx, ridx, rows, gsem, ssem):
        c = lax.axis_index("c")
        s = lax.axis_index("s")
        pltpu.sync_copy(zeros.at[pl.ds(s * rpw, rpw)], acc.at[pl.ds(s * rpw, rpw)])
        plsc.subcore_barrier()

        def run(col_h, row_h, tbl_h):
            def guard(g, fn):
                k_id = g * NS + s

                @pl.when(jnp.logical_and(k_id >= 0, k_id < nch))
                def _():
                    fn(k_id)

            def issue(g, slot):
                def f(k_id):
                    base = k_id * CH
                    pltpu.sync_copy(col_h.at[pl.ds(base, CH)], cidx.at[slot])
                    pltpu.sync_copy(row_h.at[pl.ds(base, CH)], ridx.at[slot])
                    pltpu.async_copy(tbl_h.at[cidx.at[slot]], rows.at[slot], gsem)

                guard(g, f)

            def wait_gather(g, slot):
                def f(k_id):
                    pltpu.make_async_copy(
                        tbl_h.at[cidx.at[slot]], rows.at[slot], gsem).wait()

                guard(g, f)

            def start_scatter(g, slot):
                def f(k_id):
                    pltpu.async_copy(
                        rows.at[slot], acc.at[ridx.at[slot]], ssem, add=True)

                guard(g, f)

            def drain_scatter(g, slot):
                def f(k_id):
                    pltpu.make_async_copy(
                        rows.at[slot], acc.at[ridx.at[slot]], ssem).wait()

                guard(g, f)

            issue(0, 0)

            def body(gg, carry):
                for par in (0, 1):
                    g = gg * 2 + par
                    slot, other = par, 1 - par
                    drain_scatter(g - 1, other)
                    issue(g + 1, other)
                    wait_gather(g, slot)
                    start_scatter(g, slot)
                return carry

            big_g = (niter + 1) // 2
            lax.fori_loop(0, big_g, body, 0)
            drain_scatter(2 * big_g - 1, 1)

        @pl.when(c == 0)
        def _():
            run(ca, ra, ta)

        @pl.when(c == 1)
        def _():
            run(cb, rb, tb)
        plsc.subcore_barrier()
        pltpu.sync_copy(acc.at[pl.ds(s * rpw, rpw)],
                        out.at[pl.ds(c * n_pad + s * rpw, rpw)])

    zeros = jnp.zeros((n_pad, D), jnp.float32)
    out = k(tbl_a, tbl_b, col_a, row_a, col_b, row_b, zeros)
    return out[:n_rows], out[n_pad : n_pad + n_rows]


def _rel_combine(rel_emb, col, row, val, n_rows):
    E = col.shape[0]
    NW = NC * NS
    nch = E // CH
    niter = (nch + NW - 1) // NW
    n_pad = -(-n_rows // (8 * NS)) * (8 * NS)
    rpw = n_pad // NS

    @functools.partial(
        pl.kernel,
        out_type=jax.ShapeDtypeStruct((2 * n_pad, D), jnp.float32),
        mesh=_MESH,
        scratch_types=dict(
            acc=pltpu.VMEM_SHARED((n_pad, D), jnp.float32),
            cidx=pltpu.VMEM((CH,), jnp.int32),
            ridx=pltpu.VMEM((CH,), jnp.int32),
            vals=pltpu.VMEM((CH,), jnp.float32),
            rows=pltpu.VMEM((CH, D), jnp.float32),
            sem=pltpu.SemaphoreType.DMA,
        ),
    )
    def k(tbl, ch, rh, vh, zeros, out, acc, cidx, ridx, vals, rows, sem):
        c = lax.axis_index("c")
        s = lax.axis_index("s")
        w = s * NC + c
        pltpu.sync_copy(zeros.at[pl.ds(s * rpw, rpw)], acc.at[pl.ds(s * rpw, rpw)])
        plsc.subcore_barrier()

        def body(g, carry):
            k_id = g * NW + w

            @pl.when(k_id < nch)
            def _():
                base = k_id * CH
                pltpu.sync_copy(ch.at[pl.ds(base, CH)], cidx)
                pltpu.sync_copy(rh.at[pl.ds(base, CH)], ridx)
                pltpu.sync_copy(vh.at[pl.ds(base, CH)], vals)
                pltpu.async_copy(tbl.at[cidx], rows, sem).wait()

                def mul(q, cc):
                    v16 = vals[pl.ds(q * 16, 16)]
                    for e16 in range(16):
                        v = v16[e16]
                        e = q * 16 + e16
                        for j in range(D // 16):
                            sl = pl.ds(j * 16, 16)
                            rows[e, sl] = rows[e, sl] * v
                    return cc

                lax.fori_loop(0, CH // 16, mul, 0)
                pltpu.sync_copy(rows, acc.at[ridx], add=True)

            return carry

        lax.fori_loop(0, niter, body, 0)
        plsc.subcore_barrier()
        pltpu.sync_copy(acc.at[pl.ds(s * rpw, rpw)],
                        out.at[pl.ds(c * n_pad + s * rpw, rpw)])

    zeros = jnp.zeros((n_pad, D), jnp.float32)
    out = k(rel_emb, col, row, val, zeros)
    return out[:n_rows] + out[n_pad : n_pad + n_rows]


def _degrees(ent_row, rel_row, adj_row, n_rows):
    E = ent_row.shape[0]
    NW = NC * NS
    nch = E // CH
    niter_c = (nch + NS - 1) // NS
    niter_w = (nch + NW - 1) // NW
    n_pad = -(-n_rows // (8 * NS)) * (8 * NS)
    rpw = n_pad // NS

    @functools.partial(
        pl.kernel,
        out_type=jax.ShapeDtypeStruct((4 * n_pad, D), jnp.float32),
        mesh=_MESH,
        scratch_types=dict(
            acc=pltpu.VMEM_SHARED((n_pad, D), jnp.float32),
            ridx=pltpu.VMEM((CH,), jnp.int32),
            ones_v=pltpu.VMEM((CH, D), jnp.float32),
        ),
    )
    def k(eh, rh, ah, zeros, out, acc, ridx, ones_v):
        c = lax.axis_index("c")
        s = lax.axis_index("s")
        w = s * NC + c

        def fill(r, carry):
            for j in range(D // 16):
                ones_v[r, pl.ds(j * 16, 16)] = jnp.ones((16,), jnp.float32)
            return carry

        lax.fori_loop(0, CH, fill, 0)
        pltpu.sync_copy(zeros.at[pl.ds(s * rpw, rpw)], acc.at[pl.ds(s * rpw, rpw)])
        plsc.subcore_barrier()

        def count(rows_h, stride, first):
            def body(g, carry):
                k_id = g * stride + first

                @pl.when(k_id < nch)
                def _():
                    pltpu.sync_copy(rows_h.at[pl.ds(k_id * CH, CH)], ridx)
                    pltpu.sync_copy(ones_v, acc.at[ridx], add=True)

                return carry

            lax.fori_loop(0, (nch + stride - 1) // stride, body, 0)

        @pl.when(c == 0)
        def _():
            count(eh, NS, s)

        @pl.when(c == 1)
        def _():
            count(rh, NS, s)

        plsc.subcore_barrier()
        pltpu.sync_copy(acc.at[pl.ds(s * rpw, rpw)],
                        out.at[pl.ds(c * n_pad + s * rpw, rpw)])
        pltpu.sync_copy(zeros.at[pl.ds(s * rpw, rpw)], acc.at[pl.ds(s * rpw, rpw)])
        plsc.subcore_barrier()
        count(ah, NW, w)
        plsc.subcore_barrier()
        pltpu.sync_copy(acc.at[pl.ds(s * rpw, rpw)],
                        out.at[pl.ds((2 + c) * n_pad + s * rpw, rpw)])

    zeros = jnp.zeros((n_pad, D), jnp.float32)
    out = k(ent_row, rel_row, adj_row, zeros)
    deg_ent = out[:n_rows, 0]
    deg_rel = out[n_pad : n_pad + n_rows, 0]
    deg_adj = (out[2 * n_pad : 2 * n_pad + n_rows, 0]
               + out[3 * n_pad : 3 * n_pad + n_rows, 0])
    return deg_ent, deg_rel, deg_adj


def kernel(ent_adj, rel_adj, node_size, rel_size, adj_list, r_index, r_val,
           triple_size, mask, ent_emb, rel_emb, e_kernels, r_kernels):
    N = mask.shape[0]
    NR = rel_emb.shape[0]
    E = adj_list.shape[1]

    i32 = jnp.int32
    ent_row = ent_adj[0].astype(i32)
    ent_col = ent_adj[1].astype(i32)
    rel_row = rel_adj[0].astype(i32)
    rel_col = rel_adj[1].astype(i32)
    adj_row = adj_list[0].astype(i32)
    adj_col = adj_list[1].astype(i32)
    r_seg = r_index[0].astype(i32)
    r_rel = r_index[1].astype(i32)

    deg_ent, deg_rel, deg_adj = _degrees(ent_row, rel_row, adj_row, N)

    s_ent, s_rel = _dual_segsum(ent_emb, rel_emb, ent_col, ent_row, rel_col, rel_row, N)
    ent_feature = s_ent / (deg_ent + 1e-12)[:, None]
    rel_feature = s_rel / (deg_rel + 1e-12)[:, None]

    Rm = _rel_combine(rel_emb, r_rel, r_seg, r_val, NR)
    Rn = Rm / (jnp.linalg.norm(Rm, axis=1, keepdims=True) + 1e-12)

    row_s = adj_row[:NR]
    col_s = adj_col[:NR]

    def fixup(f, S, kvec):
        a = (Rn @ kvec)[:, 0]
        Ev = jnp.exp(a)
        G = f[col_s]
        dd = jnp.sum(G * Rn, axis=1)
        reflterm = Ev[:, None] * G - (2.0 * Ev * dd)[:, None] * Rn
        P = jax.ops.segment_sum(G, row_s, num_segments=N)
        Qn = jax.ops.segment_sum(reflterm, row_s, num_segments=N)
        cnt = jax.ops.segment_sum(jnp.ones((NR,), jnp.float32), row_s, num_segments=N)
        sE = jax.ops.segment_sum(Ev, row_s, num_segments=N)
        denom = (deg_adj - cnt) + sE
        return jnp.tanh((S - P + Qn) / (denom + 1e-12)[:, None])

    f_e = jnp.tanh(ent_feature)
    f_r = jnp.tanh(rel_feature)
    outs = [f_e, None, None, f_r, None, None]
    for l in range(2):
        s_e, s_r = _dual_segsum(f_e, f_r, adj_col, adj_row, adj_col, adj_row, N)
        f_e = fixup(f_e, s_e, e_kernels[l])
        f_r = fixup(f_r, s_r, r_kernels[l])
        outs[1 + l] = f_e
        outs[4 + l] = f_r

    return jnp.concatenate(outs, axis=-1)


# speedup vs baseline: 20.5543x; 1.0575x over previous
import functools

import jax
import jax.numpy as jnp
from jax import lax
from jax.experimental import pallas as pl
from jax.experimental.pallas import tpu as pltpu
from jax.experimental.pallas import tpu_sc as plsc

NC = 2
NS = 16
CH = 128
D = 128

_MESH = plsc.VectorSubcoreMesh(core_axis_name="c", subcore_axis_name="s")


def _dual_segsum(tbl_a, tbl_b, col_a, row_a, col_b, row_b, n_rows):
    E = col_a.shape[0]
    nch = E // CH
    niter = (nch + NS - 1) // NS
    n_pad = -(-n_rows // (8 * NS)) * (8 * NS)
    rpw = n_pad // NS

    @functools.partial(
        pl.kernel,
        out_type=jax.ShapeDtypeStruct((2 * n_pad, D), jnp.float32),
        mesh=_MESH,
        scratch_types=dict(
            acc=pltpu.VMEM_SHARED((n_pad, D), jnp.float32),
            cidx=pltpu.VMEM((2, CH), jnp.int32),
            ridx=pltpu.VMEM((2, CH), jnp.int32),
            rows=pltpu.VMEM((2, CH, D), jnp.float32),
            gsem=pltpu.SemaphoreType.DMA,
            ssem=pltpu.SemaphoreType.DMA,
        ),
    )
    def k(ta, tb, ca, ra, cb, rb, zeros, out, acc, cidx, ridx, rows, gsem, ssem):
        c = lax.axis_index("c")
        s = lax.axis_index("s")
        pltpu.sync_copy(zeros.at[pl.ds(s * rpw, rpw)], acc.at[pl.ds(s * rpw, rpw)])
        plsc.subcore_barrier()

        def run(col_h, row_h, tbl_h):
            def guard(g, fn):
                k_id = g * NS + s

                @pl.when(jnp.logical_and(k_id >= 0, k_id < nch))
                def _():
                    fn(k_id)

            def issue(g, slot):
                def f(k_id):
                    base = k_id * CH
                    pltpu.sync_copy(col_h.at[pl.ds(base, CH)], cidx.at[slot])
                    pltpu.sync_copy(row_h.at[pl.ds(base, CH)], ridx.at[slot])
                    pltpu.async_copy(tbl_h.at[cidx.at[slot]], rows.at[slot], gsem)

                guard(g, f)

            def wait_gather(g, slot):
                def f(k_id):
                    pltpu.make_async_copy(
                        tbl_h.at[cidx.at[slot]], rows.at[slot], gsem).wait()

                guard(g, f)

            def start_scatter(g, slot):
                def f(k_id):
                    pltpu.async_copy(
                        rows.at[slot], acc.at[ridx.at[slot]], ssem, add=True)

                guard(g, f)

            def drain_scatter(g, slot):
                def f(k_id):
                    pltpu.make_async_copy(
                        rows.at[slot], acc.at[ridx.at[slot]], ssem).wait()

                guard(g, f)

            issue(0, 0)

            def body(gg, carry):
                for par in (0, 1):
                    g = gg * 2 + par
                    slot, other = par, 1 - par
                    drain_scatter(g - 1, other)
                    issue(g + 1, other)
                    wait_gather(g, slot)
                    start_scatter(g, slot)
                return carry

            big_g = (niter + 1) // 2
            lax.fori_loop(0, big_g, body, 0)
            drain_scatter(2 * big_g - 1, 1)

        @pl.when(c == 0)
        def _():
            run(ca, ra, ta)

        @pl.when(c == 1)
        def _():
            run(cb, rb, tb)
        plsc.subcore_barrier()
        pltpu.sync_copy(acc.at[pl.ds(s * rpw, rpw)],
                        out.at[pl.ds(c * n_pad + s * rpw, rpw)])

    zeros = jnp.zeros((n_pad, D), jnp.float32)
    out = k(tbl_a, tbl_b, col_a, row_a, col_b, row_b, zeros)
    return out[:n_rows], out[n_pad : n_pad + n_rows]


def _rel_combine(rel_emb, col, row, val, n_rows):
    E = col.shape[0]
    NW = NC * NS
    nch = E // CH
    niter = (nch + NW - 1) // NW
    n_pad = -(-n_rows // (8 * NS)) * (8 * NS)
    rpw = n_pad // NS

    @functools.partial(
        pl.kernel,
        out_type=jax.ShapeDtypeStruct((2 * n_pad, D), jnp.float32),
        mesh=_MESH,
        scratch_types=dict(
            acc=pltpu.VMEM_SHARED((n_pad, D), jnp.float32),
            cidx=pltpu.VMEM((2, CH), jnp.int32),
            ridx=pltpu.VMEM((2, CH), jnp.int32),
            vals=pltpu.VMEM((2, CH), jnp.float32),
            rows=pltpu.VMEM((2, CH, D), jnp.float32),
            gsem=pltpu.SemaphoreType.DMA,
            ssem=pltpu.SemaphoreType.DMA,
        ),
    )
    def k(tbl, ch, rh, vh, zeros, out, acc, cidx, ridx, vals, rows, gsem, ssem):
        c = lax.axis_index("c")
        s = lax.axis_index("s")
        w = s * NC + c
        pltpu.sync_copy(zeros.at[pl.ds(s * rpw, rpw)], acc.at[pl.ds(s * rpw, rpw)])
        plsc.subcore_barrier()

        def guard(g, fn):
            k_id = g * NW + w

            @pl.when(jnp.logical_and(k_id >= 0, k_id < nch))
            def _():
                fn(k_id)

        def issue(g, slot):
            def f(k_id):
                base = k_id * CH
                pltpu.sync_copy(ch.at[pl.ds(base, CH)], cidx.at[slot])
                pltpu.sync_copy(rh.at[pl.ds(base, CH)], ridx.at[slot])
                pltpu.sync_copy(vh.at[pl.ds(base, CH)], vals.at[slot])
                pltpu.async_copy(tbl.at[cidx.at[slot]], rows.at[slot], gsem)

            guard(g, f)

        def wait_scale_scatter(g, slot):
            def f(k_id):
                pltpu.make_async_copy(
                    tbl.at[cidx.at[slot]], rows.at[slot], gsem).wait()

                def mul(q, cc):
                    v16 = vals[slot, pl.ds(q * 16, 16)]
                    for e16 in range(16):
                        v = v16[e16]
                        e = q * 16 + e16
                        for j in range(D // 16):
                            sl = pl.ds(j * 16, 16)
                            rows[slot, e, sl] = rows[slot, e, sl] * v
                    return cc

                lax.fori_loop(0, CH // 16, mul, 0)
                pltpu.async_copy(rows.at[slot], acc.at[ridx.at[slot]], ssem, add=True)

            guard(g, f)

        def drain_scatter(g, slot):
            def f(k_id):
                pltpu.make_async_copy(
                    rows.at[slot], acc.at[ridx.at[slot]], ssem).wait()

            guard(g, f)

        issue(0, 0)

        def body(gg, carry):
            for par in (0, 1):
                g = gg * 2 + par
                slot, other = par, 1 - par
                drain_scatter(g - 1, other)
                issue(g + 1, other)
                wait_scale_scatter(g, slot)
            return carry

        big_g = (niter + 1) // 2
        lax.fori_loop(0, big_g, body, 0)
        drain_scatter(2 * big_g - 1, 1)
        plsc.subcore_barrier()
        pltpu.sync_copy(acc.at[pl.ds(s * rpw, rpw)],
                        out.at[pl.ds(c * n_pad + s * rpw, rpw)])

    zeros = jnp.zeros((n_pad, D), jnp.float32)
    out = k(rel_emb, col, row, val, zeros)
    return out[:n_rows] + out[n_pad : n_pad + n_rows]


def _degrees(ent_row, rel_row, adj_row, n_rows):
    E = ent_row.shape[0]
    NW = NC * NS
    nch = E // CH
    niter_c = (nch + NS - 1) // NS
    niter_w = (nch + NW - 1) // NW
    n_pad = -(-n_rows // (8 * NS)) * (8 * NS)
    rpw = n_pad // NS

    @functools.partial(
        pl.kernel,
        out_type=jax.ShapeDtypeStruct((4 * n_pad, D), jnp.float32),
        mesh=_MESH,
        scratch_types=dict(
            acc=pltpu.VMEM_SHARED((n_pad, D), jnp.float32),
            ridx=pltpu.VMEM((CH,), jnp.int32),
            ones_v=pltpu.VMEM((CH, D), jnp.float32),
        ),
    )
    def k(eh, rh, ah, zeros, out, acc, ridx, ones_v):
        c = lax.axis_index("c")
        s = lax.axis_index("s")
        w = s * NC + c

        def fill(r, carry):
            for j in range(D // 16):
                ones_v[r, pl.ds(j * 16, 16)] = jnp.ones((16,), jnp.float32)
            return carry

        lax.fori_loop(0, CH, fill, 0)
        pltpu.sync_copy(zeros.at[pl.ds(s * rpw, rpw)], acc.at[pl.ds(s * rpw, rpw)])
        plsc.subcore_barrier()

        def count(rows_h, stride, first):
            def body(g, carry):
                k_id = g * stride + first

                @pl.when(k_id < nch)
                def _():
                    pltpu.sync_copy(rows_h.at[pl.ds(k_id * CH, CH)], ridx)
                    pltpu.sync_copy(ones_v, acc.at[ridx], add=True)

                return carry

            lax.fori_loop(0, (nch + stride - 1) // stride, body, 0)

        @pl.when(c == 0)
        def _():
            count(eh, NS, s)

        @pl.when(c == 1)
        def _():
            count(rh, NS, s)

        plsc.subcore_barrier()
        pltpu.sync_copy(acc.at[pl.ds(s * rpw, rpw)],
                        out.at[pl.ds(c * n_pad + s * rpw, rpw)])
        pltpu.sync_copy(zeros.at[pl.ds(s * rpw, rpw)], acc.at[pl.ds(s * rpw, rpw)])
        plsc.subcore_barrier()
        count(ah, NW, w)
        plsc.subcore_barrier()
        pltpu.sync_copy(acc.at[pl.ds(s * rpw, rpw)],
                        out.at[pl.ds((2 + c) * n_pad + s * rpw, rpw)])

    zeros = jnp.zeros((n_pad, D), jnp.float32)
    out = k(ent_row, rel_row, adj_row, zeros)
    deg_ent = out[:n_rows, 0]
    deg_rel = out[n_pad : n_pad + n_rows, 0]
    deg_adj = (out[2 * n_pad : 2 * n_pad + n_rows, 0]
               + out[3 * n_pad : 3 * n_pad + n_rows, 0])
    return deg_ent, deg_rel, deg_adj


def kernel(ent_adj, rel_adj, node_size, rel_size, adj_list, r_index, r_val,
           triple_size, mask, ent_emb, rel_emb, e_kernels, r_kernels):
    N = mask.shape[0]
    NR = rel_emb.shape[0]
    E = adj_list.shape[1]

    i32 = jnp.int32
    ent_row = ent_adj[0].astype(i32)
    ent_col = ent_adj[1].astype(i32)
    rel_row = rel_adj[0].astype(i32)
    rel_col = rel_adj[1].astype(i32)
    adj_row = adj_list[0].astype(i32)
    adj_col = adj_list[1].astype(i32)
    r_seg = r_index[0].astype(i32)
    r_rel = r_index[1].astype(i32)

    deg_ent, deg_rel, deg_adj = _degrees(ent_row, rel_row, adj_row, N)

    s_ent, s_rel = _dual_segsum(ent_emb, rel_emb, ent_col, ent_row, rel_col, rel_row, N)
    ent_feature = s_ent / (deg_ent + 1e-12)[:, None]
    rel_feature = s_rel / (deg_rel + 1e-12)[:, None]

    Rm = _rel_combine(rel_emb, r_rel, r_seg, r_val, NR)
    Rn = Rm / (jnp.linalg.norm(Rm, axis=1, keepdims=True) + 1e-12)

    row_s = adj_row[:NR]
    col_s = adj_col[:NR]

    def fixup(f, S, kvec):
        a = (Rn @ kvec)[:, 0]
        Ev = jnp.exp(a)
        G = f[col_s]
        dd = jnp.sum(G * Rn, axis=1)
        reflterm = Ev[:, None] * G - (2.0 * Ev * dd)[:, None] * Rn
        P = jax.ops.segment_sum(G, row_s, num_segments=N)
        Qn = jax.ops.segment_sum(reflterm, row_s, num_segments=N)
        cnt = jax.ops.segment_sum(jnp.ones((NR,), jnp.float32), row_s, num_segments=N)
        sE = jax.ops.segment_sum(Ev, row_s, num_segments=N)
        denom = (deg_adj - cnt) + sE
        return jnp.tanh((S - P + Qn) / (denom + 1e-12)[:, None])

    f_e = jnp.tanh(ent_feature)
    f_r = jnp.tanh(rel_feature)
    outs = [f_e, None, None, f_r, None, None]
    for l in range(2):
        s_e, s_r = _dual_segsum(f_e, f_r, adj_col, adj_row, adj_col, adj_row, N)
        f_e = fixup(f_e, s_e, e_kernels[l])
        f_r = fixup(f_r, s_r, r_kernels[l])
        outs[1 + l] = f_e
        outs[4 + l] = f_r

    return jnp.concatenate(outs, axis=-1)


# speedup vs baseline: 21.7698x; 1.0591x over previous
import functools

import jax
import jax.numpy as jnp
from jax import lax
from jax.experimental import pallas as pl
from jax.experimental.pallas import tpu as pltpu
from jax.experimental.pallas import tpu_sc as plsc

NC = 2
NS = 16
CH = 128
D = 128

_MESH = plsc.VectorSubcoreMesh(core_axis_name="c", subcore_axis_name="s")


def _dual_segsum(tbl_a, tbl_b, col_a, row_a, col_b, row_b, n_rows):
    E = col_a.shape[0]
    nch = E // CH
    niter = (nch + NS - 1) // NS
    n_pad = -(-n_rows // (8 * NS)) * (8 * NS)
    rpw = n_pad // NS

    @functools.partial(
        pl.kernel,
        out_type=jax.ShapeDtypeStruct((2 * n_pad, D), jnp.float32),
        mesh=_MESH,
        scratch_types=dict(
            acc=pltpu.VMEM_SHARED((n_pad, D), jnp.float32),
            cidx=pltpu.VMEM((2, CH), jnp.int32),
            ridx=pltpu.VMEM((2, CH), jnp.int32),
            rows=pltpu.VMEM((2, CH, D), jnp.float32),
            gsem=pltpu.SemaphoreType.DMA,
            ssem=pltpu.SemaphoreType.DMA,
        ),
    )
    def k(ta, tb, ca, ra, cb, rb, zeros, out, acc, cidx, ridx, rows, gsem, ssem):
        c = lax.axis_index("c")
        s = lax.axis_index("s")
        pltpu.sync_copy(zeros.at[pl.ds(s * rpw, rpw)], acc.at[pl.ds(s * rpw, rpw)])
        plsc.subcore_barrier()

        def run(col_h, row_h, tbl_h):
            def guard(g, fn):
                k_id = g * NS + s

                @pl.when(jnp.logical_and(k_id >= 0, k_id < nch))
                def _():
                    fn(k_id)

            def issue(g, slot):
                def f(k_id):
                    base = k_id * CH
                    pltpu.sync_copy(col_h.at[pl.ds(base, CH)], cidx.at[slot])
                    pltpu.sync_copy(row_h.at[pl.ds(base, CH)], ridx.at[slot])
                    pltpu.async_copy(tbl_h.at[cidx.at[slot]], rows.at[slot], gsem)

                guard(g, f)

            def wait_gather(g, slot):
                def f(k_id):
                    pltpu.make_async_copy(
                        tbl_h.at[cidx.at[slot]], rows.at[slot], gsem).wait()

                guard(g, f)

            def start_scatter(g, slot):
                def f(k_id):
                    pltpu.async_copy(
                        rows.at[slot], acc.at[ridx.at[slot]], ssem, add=True)

                guard(g, f)

            def drain_scatter(g, slot):
                def f(k_id):
                    pltpu.make_async_copy(
                        rows.at[slot], acc.at[ridx.at[slot]], ssem).wait()

                guard(g, f)

            issue(0, 0)

            def body(gg, carry):
                for par in (0, 1):
                    g = gg * 2 + par
                    slot, other = par, 1 - par
                    drain_scatter(g - 1, other)
                    issue(g + 1, other)
                    wait_gather(g, slot)
                    start_scatter(g, slot)
                return carry

            big_g = (niter + 1) // 2
            lax.fori_loop(0, big_g, body, 0)
            drain_scatter(2 * big_g - 1, 1)

        @pl.when(c == 0)
        def _():
            run(ca, ra, ta)

        @pl.when(c == 1)
        def _():
            run(cb, rb, tb)
        plsc.subcore_barrier()
        pltpu.sync_copy(acc.at[pl.ds(s * rpw, rpw)],
                        out.at[pl.ds(c * n_pad + s * rpw, rpw)])

    zeros = jnp.zeros((n_pad, D), jnp.float32)
    out = k(tbl_a, tbl_b, col_a, row_a, col_b, row_b, zeros)
    return out[:n_rows], out[n_pad : n_pad + n_rows]


def _rel_combine(rel_emb, col, row, val, n_rows):
    E = col.shape[0]
    NW = NC * NS
    nch = E // CH
    niter = (nch + NW - 1) // NW
    n_pad = -(-n_rows // (8 * NS)) * (8 * NS)
    rpw = n_pad // NS

    @functools.partial(
        pl.kernel,
        out_type=jax.ShapeDtypeStruct((2 * n_pad, D), jnp.float32),
        mesh=_MESH,
        scratch_types=dict(
            acc=pltpu.VMEM_SHARED((n_pad, D), jnp.float32),
            cidx=pltpu.VMEM((2, CH), jnp.int32),
            ridx=pltpu.VMEM((2, CH), jnp.int32),
            vals=pltpu.VMEM((2, CH), jnp.float32),
            rows=pltpu.VMEM((2, CH, D), jnp.float32),
            gsem=pltpu.SemaphoreType.DMA,
            ssem=pltpu.SemaphoreType.DMA,
        ),
    )
    def k(tbl, ch, rh, vh, zeros, out, acc, cidx, ridx, vals, rows, gsem, ssem):
        c = lax.axis_index("c")
        s = lax.axis_index("s")
        w = s * NC + c
        pltpu.sync_copy(zeros.at[pl.ds(s * rpw, rpw)], acc.at[pl.ds(s * rpw, rpw)])
        plsc.subcore_barrier()

        def guard(g, fn):
            k_id = g * NW + w

            @pl.when(jnp.logical_and(k_id >= 0, k_id < nch))
            def _():
                fn(k_id)

        def issue(g, slot):
            def f(k_id):
                base = k_id * CH
                pltpu.sync_copy(ch.at[pl.ds(base, CH)], cidx.at[slot])
                pltpu.sync_copy(rh.at[pl.ds(base, CH)], ridx.at[slot])
                pltpu.sync_copy(vh.at[pl.ds(base, CH)], vals.at[slot])
                pltpu.async_copy(tbl.at[cidx.at[slot]], rows.at[slot], gsem)

            guard(g, f)

        def wait_scale_scatter(g, slot):
            def f(k_id):
                pltpu.make_async_copy(
                    tbl.at[cidx.at[slot]], rows.at[slot], gsem).wait()

                def mul(q, cc):
                    v16 = vals[slot, pl.ds(q * 16, 16)]
                    for e16 in range(16):
                        v = v16[e16]
                        e = q * 16 + e16
                        for j in range(D // 16):
                            sl = pl.ds(j * 16, 16)
                            rows[slot, e, sl] = rows[slot, e, sl] * v
                    return cc

                lax.fori_loop(0, CH // 16, mul, 0)
                pltpu.async_copy(rows.at[slot], acc.at[ridx.at[slot]], ssem, add=True)

            guard(g, f)

        def drain_scatter(g, slot):
            def f(k_id):
                pltpu.make_async_copy(
                    rows.at[slot], acc.at[ridx.at[slot]], ssem).wait()

            guard(g, f)

        issue(0, 0)

        def body(gg, carry):
            for par in (0, 1):
                g = gg * 2 + par
                slot, other = par, 1 - par
                drain_scatter(g - 1, other)
                issue(g + 1, other)
                wait_scale_scatter(g, slot)
            return carry

        big_g = (niter + 1) // 2
        lax.fori_loop(0, big_g, body, 0)
        drain_scatter(2 * big_g - 1, 1)
        plsc.subcore_barrier()
        pltpu.sync_copy(acc.at[pl.ds(s * rpw, rpw)],
                        out.at[pl.ds(c * n_pad + s * rpw, rpw)])

    zeros = jnp.zeros((n_pad, D), jnp.float32)
    out = k(rel_emb, col, row, val, zeros)
    return out[:n_rows] + out[n_pad : n_pad + n_rows]


def _degrees(ent_row, rel_row, adj_row, n_rows):
    E = ent_row.shape[0]
    NW = NC * NS
    nch = E // CH
    niter_c = (nch + NS - 1) // NS
    niter_w = (nch + NW - 1) // NW
    n_pad = -(-n_rows // (8 * NS)) * (8 * NS)
    rpw = n_pad // NS

    @functools.partial(
        pl.kernel,
        out_type=jax.ShapeDtypeStruct((4 * n_pad, D), jnp.float32),
        mesh=_MESH,
        scratch_types=dict(
            acc=pltpu.VMEM_SHARED((n_pad, D), jnp.float32),
            ridx=pltpu.VMEM((2, CH), jnp.int32),
            ones_v=pltpu.VMEM((CH, D), jnp.float32),
            ssem=pltpu.SemaphoreType.DMA,
        ),
    )
    def k(eh, rh, ah, zeros, out, acc, ridx, ones_v, ssem):
        c = lax.axis_index("c")
        s = lax.axis_index("s")
        w = s * NC + c

        def fill(r, carry):
            for j in range(D // 16):
                ones_v[r, pl.ds(j * 16, 16)] = jnp.ones((16,), jnp.float32)
            return carry

        lax.fori_loop(0, CH, fill, 0)
        pltpu.sync_copy(zeros.at[pl.ds(s * rpw, rpw)], acc.at[pl.ds(s * rpw, rpw)])
        plsc.subcore_barrier()

        def count(rows_h, stride, first):
            def guard(g, fn):
                k_id = g * stride + first

                @pl.when(jnp.logical_and(k_id >= 0, k_id < nch))
                def _():
                    fn(k_id)

            def start(g, slot):
                def f(k_id):
                    pltpu.sync_copy(rows_h.at[pl.ds(k_id * CH, CH)], ridx.at[slot])
                    pltpu.async_copy(ones_v, acc.at[ridx.at[slot]], ssem, add=True)

                guard(g, f)

            def drain(g, slot):
                def f(k_id):
                    pltpu.make_async_copy(ones_v, acc.at[ridx.at[slot]], ssem).wait()

                guard(g, f)

            def body(gg, carry):
                for par in (0, 1):
                    g = gg * 2 + par
                    drain(g - 2, par)
                    start(g, par)
                return carry

            niter_l = (nch + stride - 1) // stride
            big_g = (niter_l + 1) // 2
            lax.fori_loop(0, big_g, body, 0)
            drain(2 * big_g - 2, 0)
            drain(2 * big_g - 1, 1)

        @pl.when(c == 0)
        def _():
            count(eh, NS, s)

        @pl.when(c == 1)
        def _():
            count(rh, NS, s)

        plsc.subcore_barrier()
        pltpu.sync_copy(acc.at[pl.ds(s * rpw, rpw)],
                        out.at[pl.ds(c * n_pad + s * rpw, rpw)])
        pltpu.sync_copy(zeros.at[pl.ds(s * rpw, rpw)], acc.at[pl.ds(s * rpw, rpw)])
        plsc.subcore_barrier()
        count(ah, NW, w)
        plsc.subcore_barrier()
        pltpu.sync_copy(acc.at[pl.ds(s * rpw, rpw)],
                        out.at[pl.ds((2 + c) * n_pad + s * rpw, rpw)])

    zeros = jnp.zeros((n_pad, D), jnp.float32)
    out = k(ent_row, rel_row, adj_row, zeros)
    deg_ent = out[:n_rows, 0]
    deg_rel = out[n_pad : n_pad + n_rows, 0]
    deg_adj = (out[2 * n_pad : 2 * n_pad + n_rows, 0]
               + out[3 * n_pad : 3 * n_pad + n_rows, 0])
    return deg_ent, deg_rel, deg_adj


def kernel(ent_adj, rel_adj, node_size, rel_size, adj_list, r_index, r_val,
           triple_size, mask, ent_emb, rel_emb, e_kernels, r_kernels):
    N = mask.shape[0]
    NR = rel_emb.shape[0]
    E = adj_list.shape[1]

    i32 = jnp.int32
    ent_row = ent_adj[0].astype(i32)
    ent_col = ent_adj[1].astype(i32)
    rel_row = rel_adj[0].astype(i32)
    rel_col = rel_adj[1].astype(i32)
    adj_row = adj_list[0].astype(i32)
    adj_col = adj_list[1].astype(i32)
    r_seg = r_index[0].astype(i32)
    r_rel = r_index[1].astype(i32)

    deg_ent, deg_rel, deg_adj = _degrees(ent_row, rel_row, adj_row, N)

    s_ent, s_rel = _dual_segsum(ent_emb, rel_emb, ent_col, ent_row, rel_col, rel_row, N)
    ent_feature = s_ent / (deg_ent + 1e-12)[:, None]
    rel_feature = s_rel / (deg_rel + 1e-12)[:, None]

    Rm = _rel_combine(rel_emb, r_rel, r_seg, r_val, NR)
    Rn = Rm / (jnp.linalg.norm(Rm, axis=1, keepdims=True) + 1e-12)

    row_s = adj_row[:NR]
    col_s = adj_col[:NR]

    def fixup(f, S, kvec):
        a = (Rn @ kvec)[:, 0]
        Ev = jnp.exp(a)
        G = f[col_s]
        dd = jnp.sum(G * Rn, axis=1)
        reflterm = Ev[:, None] * G - (2.0 * Ev * dd)[:, None] * Rn
        P = jax.ops.segment_sum(G, row_s, num_segments=N)
        Qn = jax.ops.segment_sum(reflterm, row_s, num_segments=N)
        cnt = jax.ops.segment_sum(jnp.ones((NR,), jnp.float32), row_s, num_segments=N)
        sE = jax.ops.segment_sum(Ev, row_s, num_segments=N)
        denom = (deg_adj - cnt) + sE
        return jnp.tanh((S - P + Qn) / (denom + 1e-12)[:, None])

    f_e = jnp.tanh(ent_feature)
    f_r = jnp.tanh(rel_feature)
    outs = [f_e, None, None, f_r, None, None]
    for l in range(2):
        s_e, s_r = _dual_segsum(f_e, f_r, adj_col, adj_row, adj_col, adj_row, N)
        f_e = fixup(f_e, s_e, e_kernels[l])
        f_r = fixup(f_r, s_r, r_kernels[l])
        outs[1 + l] = f_e
        outs[4 + l] = f_r

    return jnp.concatenate(outs, axis=-1)
